# Initial kernel scaffold; baseline (speedup 1.0000x reference)
import functools

import jax
import jax.numpy as jnp
from jax import lax
from jax.experimental import pallas as pl
from jax.experimental.pallas import tpu as pltpu
from jax.experimental.pallas import tpu_sc as plsc

NU = 50000
NI = 50000
NN = NU + NI
D = 32
HALF = NN // 2
NC, NS = 2, 16
NW = NC * NS

SB = 1024
GB = 128
NGB = SB // GB
ACC_ROWS = 51200
ZSTRIPE = ACC_ROWS // NS
WB = HALF // NS


def _spmm_kernel(nsb):
  ept = nsb * SB
  mesh = plsc.VectorSubcoreMesh(core_axis_name="c", subcore_axis_name="s")

  @functools.partial(
      pl.kernel,
      mesh=mesh,
      out_type=jax.ShapeDtypeStruct((NN, D), jnp.float32),
      scratch_types=[
          pltpu.VMEM((SB,), jnp.int32),
          pltpu.VMEM((SB,), jnp.int32),
          pltpu.VMEM((SB,), jnp.float32),
          pltpu.VMEM((1, GB), jnp.int32),
          pltpu.VMEM((GB, D), jnp.float32),
          pltpu.VMEM_SHARED((ACC_ROWS, D), jnp.float32),
          pltpu.SemaphoreType.DMA,
      ],
  )
  def body(emb, srcs, dsts, vals, out, src_v, dst_v, val_v, dloc_v, rows_v,
           acc, gsem):
    c = lax.axis_index("c")
    s = lax.axis_index("s")
    zero16 = jnp.zeros((16,), jnp.float32)

    def zrow(i, _):
      rows_v[i, pl.ds(0, 16)] = zero16
      rows_v[i, pl.ds(16, 16)] = zero16
      return 0

    lax.fori_loop(0, GB, zrow, 0)

    def zacc(b, _):
      pltpu.sync_copy(rows_v, acc.at[pl.ds(s * ZSTRIPE + b * GB, GB)])
      return 0

    lax.fori_loop(0, ZSTRIPE // GB, zacc, 0)
    plsc.subcore_barrier()

    cbase = c * HALF

    def super_body(b, _):
      base = s * ept + b * SB
      pltpu.sync_copy(srcs.at[pl.ds(base, SB)], src_v)
      pltpu.sync_copy(dsts.at[pl.ds(base, SB)], dst_v)
      pltpu.sync_copy(vals.at[pl.ds(base, SB)], val_v)

      def gblock(j, _):
        off = j * GB
        pltpu.async_copy(emb.at[src_v.at[pl.ds(off, GB)]], rows_v, gsem).wait()
        for i in range(GB // 16):
          dv = dst_v[pl.ds(off + i * 16, 16)]
          dl = dv - cbase
          ok = (dl >= 0) & (dl < HALF)
          dloc_v[0, pl.ds(i * 16, 16)] = jnp.where(ok, dl, HALF)

        def scale(q, _):
          eb = q * 4
          for u in range(4):
            r = eb + u
            v = val_v[off + r]
            rows_v[r, pl.ds(0, 16)] = rows_v[r, pl.ds(0, 16)] * v
            rows_v[r, pl.ds(16, 16)] = rows_v[r, pl.ds(16, 16)] * v
          return 0

        lax.fori_loop(0, GB // 4, scale, 0)
        pltpu.sync_copy(rows_v, acc.at[dloc_v.at[0]], add=True)
        return 0

      lax.fori_loop(0, NGB, gblock, 0)
      return 0

    lax.fori_loop(0, nsb, super_body, 0)

    plsc.subcore_barrier()
    pltpu.sync_copy(acc.at[pl.ds(s * WB, WB)],
                    out.at[pl.ds(cbase + s * WB, WB)])

  return body


def _final_kernel(batch):
  pb = batch // NW
  mesh = plsc.VectorSubcoreMesh(core_axis_name="c", subcore_axis_name="s")

  @functools.partial(
      pl.kernel,
      mesh=mesh,
      out_type=jax.ShapeDtypeStruct((batch,), jnp.float32),
      scratch_types=[
          pltpu.VMEM((pb,), jnp.int32),
          pltpu.VMEM((pb,), jnp.int32),
          pltpu.VMEM((pb,), jnp.int32),
          pltpu.VMEM((pb, D), jnp.float32),
          pltpu.VMEM((pb, D), jnp.float32),
          pltpu.VMEM((pb, D), jnp.float32),
          pltpu.VMEM((pb,), jnp.float32),
          pltpu.SemaphoreType.DMA,
      ],
  )
  def body(ut, it, e1, e2, e3, users, items, out,
           uidx_v, iidx_v, iidx2_v, au, ai, tmp, gam, sem):
    c = lax.axis_index("c")
    s = lax.axis_index("s")
    wid = s * NC + c
    base = wid * pb
    pltpu.sync_copy(users.at[pl.ds(base, pb)], uidx_v)
    pltpu.sync_copy(items.at[pl.ds(base, pb)], iidx_v)
    for i in range(pb // 16):
      iidx2_v[pl.ds(i * 16, 16)] = iidx_v[pl.ds(i * 16, 16)] + NU

    pltpu.async_copy(ut.at[uidx_v], au, sem).wait()
    pltpu.async_copy(it.at[iidx_v], ai, sem).wait()

    for tab in (e1, e2, e3):
      pltpu.async_copy(tab.at[uidx_v], tmp, sem).wait()

      def addu(r, _):
        au[r, pl.ds(0, 16)] = au[r, pl.ds(0, 16)] + tmp[r, pl.ds(0, 16)]
        au[r, pl.ds(16, 16)] = au[r, pl.ds(16, 16)] + tmp[r, pl.ds(16, 16)]
        return 0

      lax.fori_loop(0, pb, addu, 0)
      pltpu.async_copy(tab.at[iidx2_v], tmp, sem).wait()

      def addi(r, _):
        ai[r, pl.ds(0, 16)] = ai[r, pl.ds(0, 16)] + tmp[r, pl.ds(0, 16)]
        ai[r, pl.ds(16, 16)] = ai[r, pl.ds(16, 16)] + tmp[r, pl.ds(16, 16)]
        return 0

      lax.fori_loop(0, pb, addi, 0)

    lane = lax.iota(jnp.int32, 16)
    quarter = jnp.float32(0.25)
    one = jnp.float32(1.0)

    def outer(o, _):
      def inner(k, gvec):
        r = o * 16 + k
        u0 = au[r, pl.ds(0, 16)] * quarter
        u1 = au[r, pl.ds(16, 16)] * quarter
        s0 = one / (one + jnp.exp(-u0))
        s1 = one / (one + jnp.exp(-u1))
        i0 = ai[r, pl.ds(0, 16)] * quarter
        i1 = ai[r, pl.ds(16, 16)] * quarter
        x0 = jnp.exp(i0)
        x1 = jnp.exp(i1)
        den = jnp.sum(x0) + jnp.sum(x1)
        num = jnp.sum(s0 * x0) + jnp.sum(s1 * x1)
        return jnp.where(lane == k, num / den, gvec)

      gvec = lax.fori_loop(0, 16, inner, jnp.zeros((16,), jnp.float32))
      gam[pl.ds(o * 16, 16)] = gvec
      return 0

    lax.fori_loop(0, pb // 16, outer, 0)
    pltpu.sync_copy(gam, out.at[pl.ds(base, pb)])

  return body


def kernel(users, items, user_table, item_table, edge_index, edge_vals):
  all0 = jnp.concatenate([user_table, item_table], axis=0)
  ne = edge_vals.shape[0]
  nsb = -(-ne // (NS * SB))
  pad = nsb * NS * SB - ne
  src = jnp.concatenate([edge_index[0], jnp.zeros((pad,), jnp.int32)])
  dst = jnp.concatenate([edge_index[1], jnp.zeros((pad,), jnp.int32)])
  val = jnp.concatenate([edge_vals, jnp.zeros((pad,), jnp.float32)])

  spmm = _spmm_kernel(nsb)
  e1 = spmm(all0, src, dst, val)
  e2 = spmm(e1, src, dst, val)
  e3 = spmm(e2, src, dst, val)
  fin = _final_kernel(users.shape[0])
  return fin(user_table, item_table, e1, e2, e3, users, items)


# speedup vs baseline: 6.9152x; 6.9152x over previous
import functools

import jax
import jax.numpy as jnp
from jax import lax
from jax.experimental import pallas as pl
from jax.experimental.pallas import tpu as pltpu
from jax.experimental.pallas import tpu_sc as plsc

NU = 50000
NI = 50000
NN = NU + NI
D = 32
HALF = NN // 2
NC, NS = 2, 16
NW = NC * NS

SB = 1024
GB = 128
NGB = SB // GB
ACC_ROWS = 51200
ZSTRIPE = ACC_ROWS // NS
WB = HALF // NS


def _spmm_kernel(nsb):
  ept = nsb * SB
  mesh = plsc.VectorSubcoreMesh(core_axis_name="c", subcore_axis_name="s")

  @functools.partial(
      pl.kernel,
      mesh=mesh,
      out_type=jax.ShapeDtypeStruct((NN, D), jnp.float32),
      compiler_params=pltpu.CompilerParams(
          use_tc_tiling_on_sc=False, needs_layout_passes=False),
      scratch_types=[
          pltpu.VMEM((SB,), jnp.int32),
          pltpu.VMEM((SB,), jnp.int32),
          pltpu.VMEM((SB,), jnp.float32),
          pltpu.VMEM((1, GB), jnp.int32),
          pltpu.VMEM((GB, D), jnp.float32),
          pltpu.VMEM_SHARED((ACC_ROWS, D), jnp.float32),
          pltpu.SemaphoreType.DMA,
      ],
  )
  def body(emb, srcs, dsts, vals, out, src_v, dst_v, val_v, dloc_v, rows_v,
           acc, gsem):
    c = lax.axis_index("c")
    s = lax.axis_index("s")
    zero16 = jnp.zeros((16,), jnp.float32)

    def zrow(i, _):
      rows_v[i, pl.ds(0, 16)] = zero16
      rows_v[i, pl.ds(16, 16)] = zero16
      return 0

    lax.fori_loop(0, GB, zrow, 0)

    def zacc(b, _):
      pltpu.sync_copy(rows_v, acc.at[pl.ds(s * ZSTRIPE + b * GB, GB)])
      return 0

    lax.fori_loop(0, ZSTRIPE // GB, zacc, 0)
    plsc.subcore_barrier()

    cbase = c * HALF

    def super_body(b, _):
      base = s * ept + b * SB
      pltpu.sync_copy(srcs.at[pl.ds(base, SB)], src_v)
      pltpu.sync_copy(dsts.at[pl.ds(base, SB)], dst_v)
      pltpu.sync_copy(vals.at[pl.ds(base, SB)], val_v)

      def gblock(j, _):
        off = j * GB
        pltpu.async_copy(emb.at[src_v.at[pl.ds(off, GB)]], rows_v, gsem).wait()
        for i in range(GB // 16):
          dv = dst_v[pl.ds(off + i * 16, 16)]
          dl = dv - cbase
          ok = (dl >= 0) & (dl < HALF)
          dloc_v[0, pl.ds(i * 16, 16)] = jnp.where(ok, dl, HALF)

        def scale(q, _):
          eb = q * 16
          vv = val_v[pl.ds(off + eb, 16)]
          for u in range(16):
            r = eb + u
            v = vv[u]
            rows_v[r, pl.ds(0, 16)] = rows_v[r, pl.ds(0, 16)] * v
            rows_v[r, pl.ds(16, 16)] = rows_v[r, pl.ds(16, 16)] * v
          return 0

        lax.fori_loop(0, GB // 16, scale, 0)
        pltpu.sync_copy(rows_v, acc.at[dloc_v.at[0]], add=True)
        return 0

      lax.fori_loop(0, NGB, gblock, 0)
      return 0

    lax.fori_loop(0, nsb, super_body, 0)

    plsc.subcore_barrier()
    pltpu.sync_copy(acc.at[pl.ds(s * 3128, 3080)],
                    out.at[pl.ds(cbase + s * 3128, 3080)])

    @pl.when(s < NS - 1)
    def _():
      pltpu.sync_copy(acc.at[pl.ds(s * 3128 + 3080, 48)],
                      out.at[pl.ds(cbase + s * 3128 + 3080, 48)])

  return body


def _final_kernel(batch):
  pb = batch // NW
  mesh = plsc.VectorSubcoreMesh(core_axis_name="c", subcore_axis_name="s")

  @functools.partial(
      pl.kernel,
      mesh=mesh,
      out_type=jax.ShapeDtypeStruct((batch,), jnp.float32),
      compiler_params=pltpu.CompilerParams(
          use_tc_tiling_on_sc=False, needs_layout_passes=False),
      scratch_types=[
          pltpu.VMEM((pb,), jnp.int32),
          pltpu.VMEM((pb,), jnp.int32),
          pltpu.VMEM((pb,), jnp.int32),
          pltpu.VMEM((pb, D), jnp.float32),
          pltpu.VMEM((pb, D), jnp.float32),
          pltpu.VMEM((pb, D), jnp.float32),
          pltpu.VMEM((pb,), jnp.float32),
          pltpu.SemaphoreType.DMA,
      ],
  )
  def body(ut, it, e1, e2, e3, users, items, out,
           uidx_v, iidx_v, iidx2_v, au, ai, tmp, gam, sem):
    c = lax.axis_index("c")
    s = lax.axis_index("s")
    wid = s * NC + c
    base = wid * pb
    pltpu.sync_copy(users.at[pl.ds(base, pb)], uidx_v)
    pltpu.sync_copy(items.at[pl.ds(base, pb)], iidx_v)
    for i in range(pb // 16):
      iidx2_v[pl.ds(i * 16, 16)] = iidx_v[pl.ds(i * 16, 16)] + NU

    pltpu.async_copy(ut.at[uidx_v], au, sem).wait()
    pltpu.async_copy(it.at[iidx_v], ai, sem).wait()

    for tab in (e1, e2, e3):
      pltpu.async_copy(tab.at[uidx_v], tmp, sem).wait()

      def addu(r, _):
        au[r, pl.ds(0, 16)] = au[r, pl.ds(0, 16)] + tmp[r, pl.ds(0, 16)]
        au[r, pl.ds(16, 16)] = au[r, pl.ds(16, 16)] + tmp[r, pl.ds(16, 16)]
        return 0

      lax.fori_loop(0, pb, addu, 0)
      pltpu.async_copy(tab.at[iidx2_v], tmp, sem).wait()

      def addi(r, _):
        ai[r, pl.ds(0, 16)] = ai[r, pl.ds(0, 16)] + tmp[r, pl.ds(0, 16)]
        ai[r, pl.ds(16, 16)] = ai[r, pl.ds(16, 16)] + tmp[r, pl.ds(16, 16)]
        return 0

      lax.fori_loop(0, pb, addi, 0)

    lane = lax.iota(jnp.int32, 16)
    quarter = jnp.float32(0.25)
    one = jnp.float32(1.0)

    def outer(o, _):
      def inner(k, carry):
        numvec, denvec = carry
        r = o * 16 + k
        u0 = au[r, pl.ds(0, 16)] * quarter
        u1 = au[r, pl.ds(16, 16)] * quarter
        s0 = one / (one + jnp.exp(-u0))
        s1 = one / (one + jnp.exp(-u1))
        i0 = ai[r, pl.ds(0, 16)] * quarter
        i1 = ai[r, pl.ds(16, 16)] * quarter
        x0 = jnp.exp(i0)
        x1 = jnp.exp(i1)
        den = jnp.sum(x0) + jnp.sum(x1)
        num = jnp.sum(s0 * x0) + jnp.sum(s1 * x1)
        hit = lane == k
        return (jnp.where(hit, num, numvec), jnp.where(hit, den, denvec))

      z16 = jnp.zeros((16,), jnp.float32)
      numvec, denvec = lax.fori_loop(0, 16, inner, (z16, z16 + one))
      gam[pl.ds(o * 16, 16)] = numvec / denvec
      return 0

    lax.fori_loop(0, pb // 16, outer, 0)
    pltpu.sync_copy(gam, out.at[pl.ds(base, pb)])

  return body


def kernel(users, items, user_table, item_table, edge_index, edge_vals):
  all0 = jnp.concatenate([user_table, item_table], axis=0)
  ne = edge_vals.shape[0]
  nsb = -(-ne // (NS * SB))
  pad = nsb * NS * SB - ne
  src = jnp.concatenate([edge_index[0], jnp.zeros((pad,), jnp.int32)])
  dst = jnp.concatenate([edge_index[1], jnp.zeros((pad,), jnp.int32)])
  val = jnp.concatenate([edge_vals, jnp.zeros((pad,), jnp.float32)])

  spmm = _spmm_kernel(nsb)
  e1 = spmm(all0, src, dst, val)
  e2 = spmm(e1, src, dst, val)
  e3 = spmm(e2, src, dst, val)
  fin = _final_kernel(users.shape[0])
  return fin(user_table, item_table, e1, e2, e3, users, items)


# speedup vs baseline: 7.7790x; 1.1249x over previous
import functools

import jax
import jax.numpy as jnp
from jax import lax
from jax.experimental import pallas as pl
from jax.experimental.pallas import tpu as pltpu
from jax.experimental.pallas import tpu_sc as plsc

NU = 50000
NI = 50000
NN = NU + NI
D = 32
HALF = NN // 2
NC, NS = 2, 16
NW = NC * NS

SB = 1024
GB = 128
NGB = SB // GB
NBUF = 4
ACC_ROWS = 51200
ZSTRIPE = ACC_ROWS // NS
WB = HALF // NS


def _spmm_kernel(nsb):
  ept = nsb * SB
  mesh = plsc.VectorSubcoreMesh(core_axis_name="c", subcore_axis_name="s")

  @functools.partial(
      pl.kernel,
      mesh=mesh,
      out_type=jax.ShapeDtypeStruct((NN, D), jnp.float32),
      compiler_params=pltpu.CompilerParams(
          use_tc_tiling_on_sc=False, needs_layout_passes=False),
      scratch_types=[
          pltpu.VMEM((2, SB), jnp.int32),
          pltpu.VMEM((2, SB), jnp.int32),
          pltpu.VMEM((2, SB), jnp.float32),
          pltpu.VMEM((1, GB), jnp.int32),
          pltpu.VMEM((NBUF, GB, D), jnp.float32),
          pltpu.VMEM_SHARED((ACC_ROWS, D), jnp.float32),
          pltpu.SemaphoreType.DMA,
          pltpu.SemaphoreType.DMA,
          pltpu.SemaphoreType.DMA,
          pltpu.SemaphoreType.DMA,
          pltpu.SemaphoreType.DMA,
      ],
  )
  def body(emb, srcs, dsts, vals, out, src_v, dst_v, val_v, dloc_v, rows_v,
           acc, ssem, g0, g1, g2, g3):
    gsems = (g0, g1, g2, g3)
    c = lax.axis_index("c")
    s = lax.axis_index("s")
    zero16 = jnp.zeros((16,), jnp.float32)

    def zrow(i, _):
      rows_v[0, i, pl.ds(0, 16)] = zero16
      rows_v[0, i, pl.ds(16, 16)] = zero16
      return 0

    lax.fori_loop(0, GB, zrow, 0)

    def zacc(b, _):
      pltpu.sync_copy(rows_v.at[0], acc.at[pl.ds(s * ZSTRIPE + b * GB, GB)])
      return 0

    lax.fori_loop(0, ZSTRIPE // GB, zacc, 0)
    plsc.subcore_barrier()

    cbase = c * HALF

    def stage(b, buf):
      base = s * ept + b * SB
      pltpu.async_copy(srcs.at[pl.ds(base, SB)], src_v.at[buf], ssem)
      pltpu.async_copy(dsts.at[pl.ds(base, SB)], dst_v.at[buf], ssem)
      pltpu.async_copy(vals.at[pl.ds(base, SB)], val_v.at[buf], ssem)

    stage(0, 0)

    def super_body(b, _):
      buf = lax.rem(b, 2)
      pltpu.make_async_copy(srcs.at[pl.ds(0, SB)], src_v.at[buf], ssem).wait()
      pltpu.make_async_copy(dsts.at[pl.ds(0, SB)], dst_v.at[buf], ssem).wait()
      pltpu.make_async_copy(vals.at[pl.ds(0, SB)], val_v.at[buf], ssem).wait()

      @pl.when(b < nsb - 1)
      def _():
        stage(b + 1, 1 - buf)

      sv = src_v.at[buf]
      descs = []

      def issue(j, bu):
        return pltpu.async_copy(
            emb.at[sv.at[pl.ds(j * GB, GB)]], rows_v.at[bu], gsems[bu])

      for bu in range(NBUF):
        descs.append(issue(bu, bu))

      for j in range(NGB):
        bu = j % NBUF
        descs[j].wait()
        off = j * GB
        for i in range(GB // 16):
          dv = dst_v[buf, pl.ds(off + i * 16, 16)]
          dl = dv - cbase
          ok = (dl >= 0) & (dl < HALF)
          dloc_v[0, pl.ds(i * 16, 16)] = jnp.where(ok, dl, HALF)

        def scale(q, _):
          eb = q * 16
          vv = val_v[buf, pl.ds(off + eb, 16)]
          for u in range(16):
            r = eb + u
            v = vv[u]
            rows_v[bu, r, pl.ds(0, 16)] = rows_v[bu, r, pl.ds(0, 16)] * v
            rows_v[bu, r, pl.ds(16, 16)] = rows_v[bu, r, pl.ds(16, 16)] * v
          return 0

        lax.fori_loop(0, GB // 16, scale, 0)
        pltpu.sync_copy(rows_v.at[bu], acc.at[dloc_v.at[0]], add=True)
        if j + NBUF < NGB:
          descs.append(issue(j + NBUF, bu))
      return 0

    lax.fori_loop(0, nsb, super_body, 0)

    plsc.subcore_barrier()
    pltpu.sync_copy(acc.at[pl.ds(s * 3128, 3080)],
                    out.at[pl.ds(cbase + s * 3128, 3080)])

    @pl.when(s < NS - 1)
    def _():
      pltpu.sync_copy(acc.at[pl.ds(s * 3128 + 3080, 48)],
                      out.at[pl.ds(cbase + s * 3128 + 3080, 48)])

  return body


def _final_kernel(batch):
  pb = batch // NW
  mesh = plsc.VectorSubcoreMesh(core_axis_name="c", subcore_axis_name="s")

  @functools.partial(
      pl.kernel,
      mesh=mesh,
      out_type=jax.ShapeDtypeStruct((batch,), jnp.float32),
      compiler_params=pltpu.CompilerParams(
          use_tc_tiling_on_sc=False, needs_layout_passes=False),
      scratch_types=[
          pltpu.VMEM((pb,), jnp.int32),
          pltpu.VMEM((pb,), jnp.int32),
          pltpu.VMEM((pb,), jnp.int32),
          pltpu.VMEM((pb, D), jnp.float32),
          pltpu.VMEM((pb, D), jnp.float32),
          pltpu.VMEM((pb, D), jnp.float32),
          pltpu.VMEM((pb,), jnp.float32),
          pltpu.SemaphoreType.DMA,
      ],
  )
  def body(ut, it, e1, e2, e3, users, items, out,
           uidx_v, iidx_v, iidx2_v, au, ai, tmp, gam, sem):
    c = lax.axis_index("c")
    s = lax.axis_index("s")
    wid = s * NC + c
    base = wid * pb
    pltpu.sync_copy(users.at[pl.ds(base, pb)], uidx_v)
    pltpu.sync_copy(items.at[pl.ds(base, pb)], iidx_v)
    for i in range(pb // 16):
      iidx2_v[pl.ds(i * 16, 16)] = iidx_v[pl.ds(i * 16, 16)] + NU

    pltpu.async_copy(ut.at[uidx_v], au, sem).wait()
    pltpu.async_copy(it.at[iidx_v], ai, sem).wait()

    for tab in (e1, e2, e3):
      pltpu.async_copy(tab.at[uidx_v], tmp, sem).wait()

      def addu(r, _):
        au[r, pl.ds(0, 16)] = au[r, pl.ds(0, 16)] + tmp[r, pl.ds(0, 16)]
        au[r, pl.ds(16, 16)] = au[r, pl.ds(16, 16)] + tmp[r, pl.ds(16, 16)]
        return 0

      lax.fori_loop(0, pb, addu, 0)
      pltpu.async_copy(tab.at[iidx2_v], tmp, sem).wait()

      def addi(r, _):
        ai[r, pl.ds(0, 16)] = ai[r, pl.ds(0, 16)] + tmp[r, pl.ds(0, 16)]
        ai[r, pl.ds(16, 16)] = ai[r, pl.ds(16, 16)] + tmp[r, pl.ds(16, 16)]
        return 0

      lax.fori_loop(0, pb, addi, 0)

    lane = lax.iota(jnp.int32, 16)
    quarter = jnp.float32(0.25)
    one = jnp.float32(1.0)

    def outer(o, _):
      def inner(k, carry):
        numvec, denvec = carry
        r = o * 16 + k
        u0 = au[r, pl.ds(0, 16)] * quarter
        u1 = au[r, pl.ds(16, 16)] * quarter
        s0 = one / (one + jnp.exp(-u0))
        s1 = one / (one + jnp.exp(-u1))
        i0 = ai[r, pl.ds(0, 16)] * quarter
        i1 = ai[r, pl.ds(16, 16)] * quarter
        x0 = jnp.exp(i0)
        x1 = jnp.exp(i1)
        den = jnp.sum(x0) + jnp.sum(x1)
        num = jnp.sum(s0 * x0) + jnp.sum(s1 * x1)
        hit = lane == k
        return (jnp.where(hit, num, numvec), jnp.where(hit, den, denvec))

      z16 = jnp.zeros((16,), jnp.float32)
      numvec, denvec = lax.fori_loop(0, 16, inner, (z16, z16 + one))
      gam[pl.ds(o * 16, 16)] = numvec / denvec
      return 0

    lax.fori_loop(0, pb // 16, outer, 0)
    pltpu.sync_copy(gam, out.at[pl.ds(base, pb)])

  return body


def kernel(users, items, user_table, item_table, edge_index, edge_vals):
  all0 = jnp.concatenate([user_table, item_table], axis=0)
  ne = edge_vals.shape[0]
  nsb = -(-ne // (NS * SB))
  pad = nsb * NS * SB - ne
  src = jnp.concatenate([edge_index[0], jnp.zeros((pad,), jnp.int32)])
  dst = jnp.concatenate([edge_index[1], jnp.zeros((pad,), jnp.int32)])
  val = jnp.concatenate([edge_vals, jnp.zeros((pad,), jnp.float32)])

  spmm = _spmm_kernel(nsb)
  e1 = spmm(all0, src, dst, val)
  e2 = spmm(e1, src, dst, val)
  e3 = spmm(e2, src, dst, val)
  fin = _final_kernel(users.shape[0])
  return fin(user_table, item_table, e1, e2, e3, users, items)
